# baseline (speedup 1.0000x reference)
import functools

import jax
import jax.numpy as jnp
from jax import lax
from jax.experimental import pallas as pl
from jax.experimental.pallas import tpu as pltpu
from jax.experimental.pallas import tpu_sc as plsc

_NQ = 16
_SC_ROWS = 0


def _make_sc_gather(V, D, B):
    info = plsc.get_sparse_core_info()
    NC, NS = info.num_cores, info.num_subcores
    NW = NC * NS
    assert B % (8 * NW) == 0 and D % info.num_lanes == 0
    b_per_w = B // NW
    mesh = plsc.VectorSubcoreMesh(core_axis_name="c", subcore_axis_name="s")

    @functools.partial(
        pl.kernel,
        mesh=mesh,
        out_type=jax.ShapeDtypeStruct((B, D), jnp.float32),
        scratch_types=[
            pltpu.VMEM((b_per_w,), jnp.int32),
            pltpu.VMEM((b_per_w, D), jnp.float32),
            pltpu.SemaphoreType.DMA,
        ],
    )
    def sc_kernel(idx_hbm, table_hbm, out_hbm, idx_v, rows_v, sem):
        wid = lax.axis_index("s") * NC + lax.axis_index("c")
        base = wid * b_per_w
        pltpu.sync_copy(idx_hbm.at[pl.ds(base, b_per_w)], idx_v)

        def issue(j, carry):
            vec = idx_v[pl.ds(j * 16, 16)]
            for k in range(16):
                pltpu.async_copy(
                    table_hbm.at[pl.ds(vec[k], 1)],
                    rows_v.at[pl.ds(j * 16 + k, 1)],
                    sem,
                )
            return carry

        lax.fori_loop(0, b_per_w // 16, issue, 0)
        pltpu.make_async_copy(
            table_hbm.at[pl.ds(0, b_per_w)], rows_v, sem
        ).wait()
        pltpu.sync_copy(rows_v, out_hbm.at[pl.ds(base, b_per_w)])

    return sc_kernel


def _make_tc_gather(V, D, B):
    def tc_kernel(idx_ref, table_ref, out_ref, sems):
        def body(j, carry):
            for q in range(_NQ):
                r = j * _NQ + q
                i = idx_ref[r]
                pltpu.make_async_copy(
                    table_ref.at[pl.ds(i, 1)], out_ref.at[pl.ds(r, 1)], sems[q]
                ).start()
            return carry

        lax.fori_loop(0, B // _NQ, body, 0)
        for q in range(_NQ):
            pltpu.make_async_copy(
                table_ref.at[pl.ds(0, B // _NQ)],
                out_ref.at[pl.ds(0, B // _NQ)],
                sems[q],
            ).wait()

    return pl.pallas_call(
        tc_kernel,
        out_shape=jax.ShapeDtypeStruct((B, D), jnp.float32),
        in_specs=[
            pl.BlockSpec(memory_space=pltpu.SMEM),
            pl.BlockSpec(memory_space=pl.ANY),
        ],
        out_specs=pl.BlockSpec(memory_space=pltpu.VMEM),
        scratch_shapes=[[pltpu.SemaphoreType.DMA] * _NQ],
    )


def kernel(input, embed_table):
    B = input.shape[0]
    V, D = embed_table.shape
    idx = input.astype(jnp.int32)
    if _SC_ROWS == 0:
        return _make_tc_gather(V, D, B)(idx, embed_table)
    out_sc = _make_sc_gather(V, D, _SC_ROWS)(idx[:_SC_ROWS], embed_table)
    out_tc = _make_tc_gather(V, D, B - _SC_ROWS)(idx[_SC_ROWS:], embed_table)
    return jnp.concatenate([out_sc, out_tc], axis=0)


# speedup vs baseline: 1.0336x; 1.0336x over previous
import functools

import jax
import jax.numpy as jnp
from jax import lax
from jax.experimental import pallas as pl
from jax.experimental.pallas import tpu as pltpu
from jax.experimental.pallas import tpu_sc as plsc

_NQ = 8
_SC_ROWS = 8704


def _make_sc_gather(V, D, B):
    info = plsc.get_sparse_core_info()
    NC, NS = info.num_cores, info.num_subcores
    NW = NC * NS
    assert B % (8 * NW) == 0 and D % info.num_lanes == 0
    b_per_w = B // NW
    mesh = plsc.VectorSubcoreMesh(core_axis_name="c", subcore_axis_name="s")

    @functools.partial(
        pl.kernel,
        mesh=mesh,
        out_type=jax.ShapeDtypeStruct((B, D), jnp.float32),
        scratch_types=[
            pltpu.VMEM((b_per_w,), jnp.int32),
            pltpu.VMEM((b_per_w, D), jnp.float32),
            pltpu.SemaphoreType.DMA,
        ],
        cost_estimate=pl.CostEstimate(
            flops=0, transcendentals=0, bytes_accessed=B * D * 4 * 2
        ),
    )
    def sc_kernel(idx_hbm, table_hbm, out_hbm, idx_v, rows_v, sem):
        wid = lax.axis_index("s") * NC + lax.axis_index("c")
        base = wid * b_per_w
        pltpu.sync_copy(idx_hbm.at[pl.ds(base, b_per_w)], idx_v)

        def issue(j, carry):
            vec = idx_v[pl.ds(j * 16, 16)]
            for k in range(16):
                pltpu.async_copy(
                    table_hbm.at[pl.ds(vec[k], 1)],
                    rows_v.at[pl.ds(j * 16 + k, 1)],
                    sem,
                )
            return carry

        lax.fori_loop(0, b_per_w // 16, issue, 0)
        pltpu.make_async_copy(
            table_hbm.at[pl.ds(0, b_per_w)], rows_v, sem
        ).wait()
        pltpu.sync_copy(rows_v, out_hbm.at[pl.ds(base, b_per_w)])

    return sc_kernel


def _make_tc_gather(V, D, B):
    def tc_kernel(idx_ref, table_ref, out_ref, sems):
        def body(j, carry):
            for q in range(_NQ):
                r = j * _NQ + q
                i = idx_ref[r]
                pltpu.make_async_copy(
                    table_ref.at[pl.ds(i, 1)], out_ref.at[pl.ds(r, 1)], sems[q]
                ).start()
            return carry

        lax.fori_loop(0, B // _NQ, body, 0)
        for q in range(_NQ):
            pltpu.make_async_copy(
                table_ref.at[pl.ds(0, B // _NQ)],
                out_ref.at[pl.ds(0, B // _NQ)],
                sems[q],
            ).wait()

    return pl.pallas_call(
        tc_kernel,
        out_shape=jax.ShapeDtypeStruct((B, D), jnp.float32),
        in_specs=[
            pl.BlockSpec(memory_space=pltpu.SMEM),
            pl.BlockSpec(memory_space=pl.ANY),
        ],
        out_specs=pl.BlockSpec(memory_space=pltpu.VMEM),
        scratch_shapes=[[pltpu.SemaphoreType.DMA] * _NQ],
    )


def kernel(input, embed_table):
    B = input.shape[0]
    V, D = embed_table.shape
    idx = input.astype(jnp.int32)
    out_sc = _make_sc_gather(V, D, _SC_ROWS)(idx[:_SC_ROWS], embed_table)
    out_tc = _make_tc_gather(V, D, B - _SC_ROWS)(idx[_SC_ROWS:], embed_table)
    return jnp.concatenate([out_sc, out_tc], axis=0)
